# baseline (speedup 1.0000x reference)
import functools

import jax
import jax.numpy as jnp
from jax import lax
from jax.experimental import pallas as pl
from jax.experimental.pallas import tpu as pltpu
from jax.experimental.pallas import tpu_sc as plsc

B = 16384
DIALECT_VOCAB = 1000
CHAR_VOCAB = 100000
EMB = 64
LANES = 16
ROW = 128
NC = 2
NS = 16
NW = NC * NS
B_PER_W = B // NW
GCHUNK = 16

RL_BLOCK = 10240
PACKED_ROWS = 51200
N_SRC_BLOCKS = (CHAR_VOCAB + 1 + RL_BLOCK - 1) // RL_BLOCK


def _relayout_body(lo_ref, hi_ref, out_ref):
    out_ref[...] = jnp.concatenate([lo_ref[...].T, hi_ref[...].T], axis=1)


def _relayout(tab_t):
    grid = (PACKED_ROWS // RL_BLOCK,)
    nhb = PACKED_ROWS // RL_BLOCK
    return pl.pallas_call(
        _relayout_body,
        grid=grid,
        in_specs=[
            pl.BlockSpec((EMB, RL_BLOCK), lambda i: (0, i)),
            pl.BlockSpec((EMB, RL_BLOCK),
                         lambda i: (0, jnp.minimum(nhb + i, N_SRC_BLOCKS - 1))),
        ],
        out_specs=pl.BlockSpec((RL_BLOCK, ROW), lambda i: (i, 0)),
        out_shape=jax.ShapeDtypeStruct((PACKED_ROWS, ROW), jnp.float32),
    )(tab_t, tab_t)


def _char_gather_body(c_idx_hbm, ctab_hbm, out_hbm, idx_c, rows_c, sem_c):
    wid = lax.axis_index("s") * NC + lax.axis_index("c")
    base = wid * B_PER_W

    pltpu.sync_copy(c_idx_hbm.at[pl.ds(base, B_PER_W)], idx_c)

    def remap(i, _):
        s = pl.ds(i * LANES, LANES)
        vc = idx_c[s]
        vc = jnp.where(vc >= 0, vc, CHAR_VOCAB)
        idx_c[s] = jnp.where(vc < PACKED_ROWS, vc, vc - PACKED_ROWS)
        return 0

    lax.fori_loop(0, B_PER_W // LANES, remap, 0)

    handles = [
        pltpu.async_copy(ctab_hbm.at[idx_c[pl.ds(j * GCHUNK, GCHUNK)]],
                         rows_c.at[pl.ds(j * GCHUNK, GCHUNK)], sem_c)
        for j in range(B_PER_W // GCHUNK)
    ]
    for h in handles:
        h.wait()
    pltpu.sync_copy(rows_c, out_hbm.at[pl.ds(base, B_PER_W)])


_char_gather = functools.partial(
    pl.kernel,
    mesh=plsc.VectorSubcoreMesh(
        core_axis_name="c", subcore_axis_name="s",
        num_cores=NC, num_subcores=NS),
    out_type=jax.ShapeDtypeStruct((B, ROW), jnp.float32),
    scratch_types=[
        pltpu.VMEM((B_PER_W,), jnp.int32),
        pltpu.VMEM((B_PER_W, ROW), jnp.float32),
        pltpu.SemaphoreType.DMA,
    ],
    compiler_params=pltpu.CompilerParams(use_tc_tiling_on_sc=True),
)(_char_gather_body)


def _demb_body(d_idx_ref, dtab_ref, out_ref):
    br = out_ref.shape[0]
    di = d_idx_ref[...].reshape(br, 1)
    di = jnp.where(di >= 0, di, DIALECT_VOCAB)
    onehot = (di == lax.broadcasted_iota(jnp.int32, (br, DIALECT_VOCAB + 1),
                                         1)).astype(jnp.float32)
    out_ref[...] = jnp.dot(onehot, dtab_ref[...],
                           preferred_element_type=jnp.float32)


def _demb(d_idx, dtab, block_rows=2048):
    return pl.pallas_call(
        _demb_body,
        grid=(B // block_rows,),
        in_specs=[
            pl.BlockSpec((block_rows,), lambda i: (i,)),
            pl.BlockSpec((DIALECT_VOCAB + 1, EMB), lambda i: (0, 0)),
        ],
        out_specs=pl.BlockSpec((block_rows, EMB), lambda i: (i, 0)),
        out_shape=jax.ShapeDtypeStruct((B, EMB), jnp.float32),
    )(d_idx, dtab)


def _decode_body(demb_ref, c_idx_ref, crows_ref, w0_ref, b0_ref,
                 w1_ref, b1_ref, w2_ref, b2_ref, l0_ref, l1_ref, l2_ref):
    br = crows_ref.shape[0]
    ci = c_idx_ref[...].reshape(br, 1)
    chalf = jnp.where(ci >= 0, ci, CHAR_VOCAB) >= PACKED_ROWS
    e = demb_ref[...] + jnp.where(chalf, crows_ref[:, EMB:],
                                  crows_ref[:, :EMB])
    et = e.T
    l0_ref[...] = jnp.dot(w0_ref[...].T, et,
                          preferred_element_type=jnp.float32) + b0_ref[...]
    l1_ref[...] = jnp.dot(w1_ref[...].T, et,
                          preferred_element_type=jnp.float32) + b1_ref[...]
    l2_ref[...] = jnp.dot(w2_ref[...].T, et,
                          preferred_element_type=jnp.float32) + b2_ref[...]


def _decode(demb, c_idx, crows, W0, b0, W1, b1, W2, b2,
            block_rows=8192):
    grid = (B // block_rows,)
    n0, n1, n2 = W0.shape[1], W1.shape[1], W2.shape[1]
    full = lambda shape: pl.BlockSpec(shape, lambda i: (0, 0))
    l0t, l1t, l2t = pl.pallas_call(
        _decode_body,
        grid=grid,
        in_specs=[
            pl.BlockSpec((block_rows, EMB), lambda i: (i, 0)),
            pl.BlockSpec((block_rows,), lambda i: (i,)),
            pl.BlockSpec((block_rows, ROW), lambda i: (i, 0)),
            full((EMB, n0)), full((n0, 1)),
            full((EMB, n1)), full((n1, 1)),
            full((EMB, n2)), full((n2, 1)),
        ],
        out_specs=[
            pl.BlockSpec((n0, block_rows), lambda i: (0, i)),
            pl.BlockSpec((n1, block_rows), lambda i: (0, i)),
            pl.BlockSpec((n2, block_rows), lambda i: (0, i)),
        ],
        out_shape=[
            jax.ShapeDtypeStruct((n0, B), jnp.float32),
            jax.ShapeDtypeStruct((n1, B), jnp.float32),
            jax.ShapeDtypeStruct((n2, B), jnp.float32),
        ],
    )(demb, c_idx, crows, W0, b0.reshape(n0, 1), W1,
      b1.reshape(n1, 1), W2, b2.reshape(n2, 1))
    return l0t.T, l1t.T, l2t.T


def kernel(dialects, chars, dialect_table, char_table, W0, b0, W1, b1, W2, b2):
    c_idx = chars.reshape(B).astype(jnp.int32)
    ctab_rm = _relayout(char_table.T)
    crows = _char_gather(c_idx, ctab_rm)
    demb = _demb(dialects.reshape(B).astype(jnp.int32), dialect_table)
    return _decode(demb, c_idx, crows, W0, b0, W1, b1, W2, b2)


# speedup vs baseline: 1.0117x; 1.0117x over previous
import functools

import jax
import jax.numpy as jnp
from jax import lax
from jax.experimental import pallas as pl
from jax.experimental.pallas import tpu as pltpu
from jax.experimental.pallas import tpu_sc as plsc

B = 16384
DIALECT_VOCAB = 1000
CHAR_VOCAB = 100000
EMB = 64
LANES = 16
ROW = 128
NC = 2
NS = 16
NW = NC * NS
B_PER_W = B // NW
GCHUNK = 16

RL_BLOCK = 10240
PACKED_ROWS = 51200
N_SRC_BLOCKS = (CHAR_VOCAB + 1 + RL_BLOCK - 1) // RL_BLOCK


def _relayout_body(lo_ref, hi_ref, out_ref):
    out_ref[...] = jnp.concatenate([lo_ref[...].T, hi_ref[...].T], axis=1)


def _relayout(tab_t):
    grid = (PACKED_ROWS // RL_BLOCK,)
    nhb = PACKED_ROWS // RL_BLOCK
    return pl.pallas_call(
        _relayout_body,
        grid=grid,
        in_specs=[
            pl.BlockSpec((EMB, RL_BLOCK), lambda i: (0, i)),
            pl.BlockSpec((EMB, RL_BLOCK),
                         lambda i: (0, jnp.minimum(nhb + i, N_SRC_BLOCKS - 1))),
        ],
        out_specs=pl.BlockSpec((RL_BLOCK, ROW), lambda i: (i, 0)),
        out_shape=jax.ShapeDtypeStruct((PACKED_ROWS, ROW), jnp.float32),
    )(tab_t, tab_t)


def _char_gather_body(c_idx_hbm, ctab_hbm, out_hbm, idx_c, rows_c, sem_c):
    wid = lax.axis_index("s") * NC + lax.axis_index("c")
    base = wid * B_PER_W

    pltpu.sync_copy(c_idx_hbm.at[pl.ds(base, B_PER_W)], idx_c)

    def remap(i, _):
        s = pl.ds(i * LANES, LANES)
        vc = idx_c[s]
        vc = jnp.where(vc >= 0, vc, CHAR_VOCAB)
        idx_c[s] = jnp.where(vc < PACKED_ROWS, vc, vc - PACKED_ROWS)
        return 0

    lax.fori_loop(0, B_PER_W // LANES, remap, 0)

    handles = [
        pltpu.async_copy(ctab_hbm.at[idx_c[pl.ds(j * GCHUNK, GCHUNK)]],
                         rows_c.at[pl.ds(j * GCHUNK, GCHUNK)], sem_c)
        for j in range(B_PER_W // GCHUNK)
    ]
    for h in handles:
        h.wait()
    pltpu.sync_copy(rows_c, out_hbm.at[pl.ds(base, B_PER_W)])


_char_gather = functools.partial(
    pl.kernel,
    mesh=plsc.VectorSubcoreMesh(
        core_axis_name="c", subcore_axis_name="s",
        num_cores=NC, num_subcores=NS),
    out_type=jax.ShapeDtypeStruct((B, ROW), jnp.float32),
    scratch_types=[
        pltpu.VMEM((B_PER_W,), jnp.int32),
        pltpu.VMEM((B_PER_W, ROW), jnp.float32),
        pltpu.SemaphoreType.DMA,
    ],
    compiler_params=pltpu.CompilerParams(use_tc_tiling_on_sc=True),
)(_char_gather_body)


def _demb_body(d_idx_ref, dtab_ref, out_ref):
    br = out_ref.shape[0]
    di = d_idx_ref[...].reshape(br, 1)
    di = jnp.where(di >= 0, di, DIALECT_VOCAB)
    onehot = (di == lax.broadcasted_iota(jnp.int32, (br, DIALECT_VOCAB + 1),
                                         1)).astype(jnp.float32)
    out_ref[...] = jnp.dot(onehot, dtab_ref[...],
                           preferred_element_type=jnp.float32)


def _demb(d_idx, dtab, block_rows=2048):
    return pl.pallas_call(
        _demb_body,
        grid=(B // block_rows,),
        in_specs=[
            pl.BlockSpec((block_rows,), lambda i: (i,)),
            pl.BlockSpec((DIALECT_VOCAB + 1, EMB), lambda i: (0, 0)),
        ],
        out_specs=pl.BlockSpec((block_rows, EMB), lambda i: (i, 0)),
        out_shape=jax.ShapeDtypeStruct((B, EMB), jnp.float32),
    )(d_idx, dtab)


def _decode_body(demb_ref, c_idx_ref, crows_ref, w0_ref, b0_ref,
                 w1_ref, b1_ref, w2_ref, b2_ref, l0_ref, l1_ref, l2_ref):
    br = crows_ref.shape[0]
    ci = c_idx_ref[...].reshape(br, 1)
    chalf = jnp.where(ci >= 0, ci, CHAR_VOCAB) >= PACKED_ROWS
    e = demb_ref[...] + jnp.where(chalf, crows_ref[:, EMB:],
                                  crows_ref[:, :EMB])
    et = e.T
    l0_ref[...] = jnp.dot(w0_ref[...].T, et,
                          preferred_element_type=jnp.float32) + b0_ref[...]
    l1_ref[...] = jnp.dot(w1_ref[...].T, et,
                          preferred_element_type=jnp.float32) + b1_ref[...]
    l2_ref[...] = jnp.dot(w2_ref[...].T, et,
                          preferred_element_type=jnp.float32) + b2_ref[...]


def _decode(demb, c_idx, crows, W0, b0, W1, b1, W2, b2,
            block_rows=4096):
    grid = (B // block_rows,)
    n0, n1, n2 = W0.shape[1], W1.shape[1], W2.shape[1]
    full = lambda shape: pl.BlockSpec(shape, lambda i: (0, 0))
    l0t, l1t, l2t = pl.pallas_call(
        _decode_body,
        grid=grid,
        in_specs=[
            pl.BlockSpec((block_rows, EMB), lambda i: (i, 0)),
            pl.BlockSpec((block_rows,), lambda i: (i,)),
            pl.BlockSpec((block_rows, ROW), lambda i: (i, 0)),
            full((EMB, n0)), full((n0, 1)),
            full((EMB, n1)), full((n1, 1)),
            full((EMB, n2)), full((n2, 1)),
        ],
        out_specs=[
            pl.BlockSpec((n0, block_rows), lambda i: (0, i)),
            pl.BlockSpec((n1, block_rows), lambda i: (0, i)),
            pl.BlockSpec((n2, block_rows), lambda i: (0, i)),
        ],
        out_shape=[
            jax.ShapeDtypeStruct((n0, B), jnp.float32),
            jax.ShapeDtypeStruct((n1, B), jnp.float32),
            jax.ShapeDtypeStruct((n2, B), jnp.float32),
        ],
    )(demb, c_idx, crows, W0, b0.reshape(n0, 1), W1,
      b1.reshape(n1, 1), W2, b2.reshape(n2, 1))
    return l0t.T, l1t.T, l2t.T


def kernel(dialects, chars, dialect_table, char_table, W0, b0, W1, b1, W2, b2):
    c_idx = chars.reshape(B).astype(jnp.int32)
    ctab_rm = _relayout(char_table.T)
    crows = _char_gather(c_idx, ctab_rm)
    demb = _demb(dialects.reshape(B).astype(jnp.int32), dialect_table)
    return _decode(demb, c_idx, crows, W0, b0, W1, b1, W2, b2)
